# Initial kernel scaffold; baseline (speedup 1.0000x reference)
import jax
import jax.numpy as jnp
from jax.experimental import pallas as pl


def kernel(y_pred, y_true, group_ids):
    raise NotImplementedError("write your pallas kernel here")


# speedup vs baseline: 1.2067x; 1.2067x over previous
import numpy as np
import jax
import jax.numpy as jnp
from jax import lax
from jax.experimental import pallas as pl
from jax.experimental.pallas import tpu as pltpu

_LANES = 128


def _logaddexp(a, b):
    m = jnp.maximum(a, b)
    return m + jnp.log(1.0 + jnp.exp(-jnp.abs(a - b)))


def _kernel_body(d_tab, m_tab, yp_ref, yt_ref, gid_ref, out_ref, accum):
    step = pl.program_id(0)
    num_steps = pl.num_programs(0)
    R = gid_ref.shape[0]

    row_iota = lax.broadcasted_iota(jnp.int32, (R, _LANES), 0)
    lane_iota = lax.broadcasted_iota(jnp.int32, (R, _LANES), 1)
    idx_flat = row_iota * _LANES + lane_iota

    gid = gid_ref[...]
    yt = yt_ref[0]
    s = yp_ref[0]

    bits = lax.bitcast_convert_type(yt, jnp.int32)
    flip = jnp.where(bits < 0, jnp.int32(-1), jnp.int32(-2147483648))
    u = bits ^ flip
    high24 = lax.shift_right_logical(u, 8)
    w1 = lax.shift_left(gid, 24) | high24
    w2 = lax.shift_left(u & jnp.int32(0xFF), 15) | idx_flat

    n_pass = d_tab.shape[0]

    def exchange(i, carry):
        w1c, w2c, sc = carry
        d = d_tab[i]
        m = m_tab[i]
        bit = (idx_flat & d) != 0

        def lane_partners(ops):
            d_f = _LANES - d
            return tuple(
                jnp.where(
                    bit,
                    pltpu.roll(x, d, axis=1),
                    pltpu.roll(x, d_f, axis=1),
                )
                for x in ops
            )

        def sub_partners(ops):
            dr = lax.shift_right_logical(d, 7)
            dr_f = R - dr
            return tuple(
                jnp.where(
                    bit,
                    pltpu.roll(x, dr, axis=0),
                    pltpu.roll(x, dr_f, axis=0),
                )
                for x in ops
            )

        p1, p2, ps = lax.cond(
            d < _LANES, lane_partners, sub_partners, (w1c, w2c, sc)
        )

        self_lt = (w1c < p1) | ((w1c == p1) & (w2c < p2))
        asc = (idx_flat & m) == 0
        want_min = asc != bit
        keep = self_lt == want_min
        return (
            jnp.where(keep, w1c, p1),
            jnp.where(keep, w2c, p2),
            jnp.where(keep, sc, ps),
        )

    w1, w2, s = lax.fori_loop(0, n_pass, exchange, (w1, w2, s), unroll=4)

    sgid = lax.shift_right_logical(w1, 24)
    v = s
    for t in range(15):
        sh = 1 << t
        if sh < _LANES:
            pv = pltpu.roll(v, _LANES - sh, axis=1)
            pv = jnp.where(
                lane_iota < _LANES - sh, pv, pltpu.roll(pv, R - 1, axis=0)
            )
            pg = pltpu.roll(sgid, _LANES - sh, axis=1)
            pg = jnp.where(
                lane_iota < _LANES - sh, pg, pltpu.roll(pg, R - 1, axis=0)
            )
            valid = ~((row_iota == R - 1) & (lane_iota >= _LANES - sh))
        else:
            rs = sh // _LANES
            pv = pltpu.roll(v, R - rs, axis=0)
            pg = pltpu.roll(sgid, R - rs, axis=0)
            valid = row_iota < R - rs
        m_ok = valid & (pg == sgid)
        v = jnp.where(m_ok, _logaddexp(v, pv), v)

    contrib = s - v

    @pl.when(step == 0)
    def _():
        accum[...] = contrib

    @pl.when(step != 0)
    def _():
        accum[...] = accum[...] + contrib

    @pl.when(step == num_steps - 1)
    def _():
        A = accum[...]
        T = A
        C = jnp.full((R, _LANES), 1.0, dtype=jnp.float32)
        for t in range(15):
            sh = 1 << t
            if sh < _LANES:
                pT = pltpu.roll(T, _LANES - sh, axis=1)
                pT = jnp.where(
                    lane_iota < _LANES - sh, pT, pltpu.roll(pT, R - 1, axis=0)
                )
                pC = pltpu.roll(C, _LANES - sh, axis=1)
                pC = jnp.where(
                    lane_iota < _LANES - sh, pC, pltpu.roll(pC, R - 1, axis=0)
                )
                pg = pltpu.roll(gid, _LANES - sh, axis=1)
                pg = jnp.where(
                    lane_iota < _LANES - sh, pg, pltpu.roll(pg, R - 1, axis=0)
                )
                valid = ~((row_iota == R - 1) & (lane_iota >= _LANES - sh))
            else:
                rs = sh // _LANES
                pT = pltpu.roll(T, R - rs, axis=0)
                pC = pltpu.roll(C, R - rs, axis=0)
                pg = pltpu.roll(gid, R - rs, axis=0)
                valid = row_iota < R - rs
            m_ok = valid & (pg == gid)
            T = jnp.where(m_ok, T + pT, T)
            C = jnp.where(m_ok, C + pC, C)

        prev = pltpu.roll(gid, 1, axis=1)
        prev = jnp.where(lane_iota > 0, prev, pltpu.roll(prev, 1, axis=0))
        is_start = (idx_flat == 0) | (gid != prev)
        denom_cols = jnp.float32(num_steps)
        group_mean = jnp.where(is_start, T / (C * denom_cols), 0.0)
        n_groups = jnp.sum(jnp.where(is_start, 1.0, 0.0))
        out_ref[0, 0] = -jnp.sum(group_mean) / n_groups


def kernel(y_pred, y_true, group_ids):
    N, D = y_pred.shape
    R = N // _LANES

    ds, ms = [], []
    m = 2
    while m <= N:
        d = m // 2
        while d >= 1:
            ds.append(d)
            ms.append(m)
            d //= 2
        m *= 2
    d_tab = jnp.asarray(np.array(ds, dtype=np.int32))
    m_tab = jnp.asarray(np.array(ms, dtype=np.int32))

    ypT = (-y_pred).T.reshape(D, R, _LANES)
    ytT = y_true.T.reshape(D, R, _LANES)
    gid2 = group_ids.astype(jnp.int32).reshape(R, _LANES)

    out = pl.pallas_call(
        _kernel_body,
        grid=(D,),
        in_specs=[
            pl.BlockSpec(memory_space=pltpu.SMEM),
            pl.BlockSpec(memory_space=pltpu.SMEM),
            pl.BlockSpec((1, R, _LANES), lambda d: (d, 0, 0)),
            pl.BlockSpec((1, R, _LANES), lambda d: (d, 0, 0)),
            pl.BlockSpec((R, _LANES), lambda d: (0, 0)),
        ],
        out_specs=pl.BlockSpec(memory_space=pltpu.SMEM),
        out_shape=jax.ShapeDtypeStruct((1, 1), jnp.float32),
        scratch_shapes=[pltpu.VMEM((R, _LANES), jnp.float32)],
        compiler_params=pltpu.CompilerParams(
            dimension_semantics=("arbitrary",)
        ),
    )(d_tab, m_tab, ypT, ytT, gid2)
    return out.reshape(())


# speedup vs baseline: 1.2165x; 1.0081x over previous
import numpy as np
import jax
import jax.numpy as jnp
from jax import lax
from jax.experimental import pallas as pl
from jax.experimental.pallas import tpu as pltpu

_LANES = 128


def _logaddexp(a, b):
    m = jnp.maximum(a, b)
    return m + jnp.log(1.0 + jnp.exp(-jnp.abs(a - b)))


def _kernel_body(d_tab, m_tab, yp_ref, yt_ref, gid_ref, out_ref, accum):
    step = pl.program_id(0)
    num_steps = pl.num_programs(0)
    R = gid_ref.shape[0]

    row_iota = lax.broadcasted_iota(jnp.int32, (R, _LANES), 0)
    lane_iota = lax.broadcasted_iota(jnp.int32, (R, _LANES), 1)
    idx_flat = row_iota * _LANES + lane_iota

    gid = gid_ref[...]
    yt = yt_ref[0]
    s = yp_ref[0]

    bits = lax.bitcast_convert_type(yt, jnp.int32)
    flip = jnp.where(bits < 0, jnp.int32(-1), jnp.int32(-2147483648))
    u = bits ^ flip
    high24 = lax.shift_right_logical(u, 8)
    w1 = lax.shift_left(gid, 24) | high24
    w2 = lax.shift_left(u & jnp.int32(0xFF), 15) | idx_flat

    n_pass = d_tab.shape[0]

    def exchange(i, carry):
        w1c, w2c, sc = carry
        d = d_tab[i]
        m = m_tab[i]
        bit = (idx_flat & d) != 0

        def lane_partners(ops):
            d_f = _LANES - d
            return tuple(
                jnp.where(
                    bit,
                    pltpu.roll(x, d, axis=1),
                    pltpu.roll(x, d_f, axis=1),
                )
                for x in ops
            )

        def sub_partners(ops):
            dr = lax.shift_right_logical(d, 7)
            dr_f = R - dr
            return tuple(
                jnp.where(
                    bit,
                    pltpu.roll(x, dr, axis=0),
                    pltpu.roll(x, dr_f, axis=0),
                )
                for x in ops
            )

        p1, p2, ps = lax.cond(
            d < _LANES, lane_partners, sub_partners, (w1c, w2c, sc)
        )

        self_lt = (w1c < p1) | ((w1c == p1) & (w2c < p2))
        asc = (idx_flat & m) == 0
        want_min = asc != bit
        keep = self_lt == want_min
        return (
            jnp.where(keep, w1c, p1),
            jnp.where(keep, w2c, p2),
            jnp.where(keep, sc, ps),
        )

    w1, w2, s = lax.fori_loop(0, n_pass, exchange, (w1, w2, s), unroll=8)

    sgid = lax.shift_right_logical(w1, 24)
    v = s
    for t in range(15):
        sh = 1 << t
        if sh < _LANES:
            pv = pltpu.roll(v, _LANES - sh, axis=1)
            pv = jnp.where(
                lane_iota < _LANES - sh, pv, pltpu.roll(pv, R - 1, axis=0)
            )
            pg = pltpu.roll(sgid, _LANES - sh, axis=1)
            pg = jnp.where(
                lane_iota < _LANES - sh, pg, pltpu.roll(pg, R - 1, axis=0)
            )
            valid = ~((row_iota == R - 1) & (lane_iota >= _LANES - sh))
        else:
            rs = sh // _LANES
            pv = pltpu.roll(v, R - rs, axis=0)
            pg = pltpu.roll(sgid, R - rs, axis=0)
            valid = row_iota < R - rs
        m_ok = valid & (pg == sgid)
        v = jnp.where(m_ok, _logaddexp(v, pv), v)

    contrib = s - v

    @pl.when(step == 0)
    def _():
        accum[...] = contrib

    @pl.when(step != 0)
    def _():
        accum[...] = accum[...] + contrib

    @pl.when(step == num_steps - 1)
    def _():
        A = accum[...]
        T = A
        C = jnp.full((R, _LANES), 1.0, dtype=jnp.float32)
        for t in range(15):
            sh = 1 << t
            if sh < _LANES:
                pT = pltpu.roll(T, _LANES - sh, axis=1)
                pT = jnp.where(
                    lane_iota < _LANES - sh, pT, pltpu.roll(pT, R - 1, axis=0)
                )
                pC = pltpu.roll(C, _LANES - sh, axis=1)
                pC = jnp.where(
                    lane_iota < _LANES - sh, pC, pltpu.roll(pC, R - 1, axis=0)
                )
                pg = pltpu.roll(gid, _LANES - sh, axis=1)
                pg = jnp.where(
                    lane_iota < _LANES - sh, pg, pltpu.roll(pg, R - 1, axis=0)
                )
                valid = ~((row_iota == R - 1) & (lane_iota >= _LANES - sh))
            else:
                rs = sh // _LANES
                pT = pltpu.roll(T, R - rs, axis=0)
                pC = pltpu.roll(C, R - rs, axis=0)
                pg = pltpu.roll(gid, R - rs, axis=0)
                valid = row_iota < R - rs
            m_ok = valid & (pg == gid)
            T = jnp.where(m_ok, T + pT, T)
            C = jnp.where(m_ok, C + pC, C)

        prev = pltpu.roll(gid, 1, axis=1)
        prev = jnp.where(lane_iota > 0, prev, pltpu.roll(prev, 1, axis=0))
        is_start = (idx_flat == 0) | (gid != prev)
        denom_cols = jnp.float32(num_steps)
        group_mean = jnp.where(is_start, T / (C * denom_cols), 0.0)
        n_groups = jnp.sum(jnp.where(is_start, 1.0, 0.0))
        out_ref[0, 0] = -jnp.sum(group_mean) / n_groups


def kernel(y_pred, y_true, group_ids):
    N, D = y_pred.shape
    R = N // _LANES

    ds, ms = [], []
    m = 2
    while m <= N:
        d = m // 2
        while d >= 1:
            ds.append(d)
            ms.append(m)
            d //= 2
        m *= 2
    d_tab = jnp.asarray(np.array(ds, dtype=np.int32))
    m_tab = jnp.asarray(np.array(ms, dtype=np.int32))

    ypT = (-y_pred).T.reshape(D, R, _LANES)
    ytT = y_true.T.reshape(D, R, _LANES)
    gid2 = group_ids.astype(jnp.int32).reshape(R, _LANES)

    out = pl.pallas_call(
        _kernel_body,
        grid=(D,),
        in_specs=[
            pl.BlockSpec(memory_space=pltpu.SMEM),
            pl.BlockSpec(memory_space=pltpu.SMEM),
            pl.BlockSpec((1, R, _LANES), lambda d: (d, 0, 0)),
            pl.BlockSpec((1, R, _LANES), lambda d: (d, 0, 0)),
            pl.BlockSpec((R, _LANES), lambda d: (0, 0)),
        ],
        out_specs=pl.BlockSpec(memory_space=pltpu.SMEM),
        out_shape=jax.ShapeDtypeStruct((1, 1), jnp.float32),
        scratch_shapes=[pltpu.VMEM((R, _LANES), jnp.float32)],
        compiler_params=pltpu.CompilerParams(
            dimension_semantics=("arbitrary",)
        ),
    )(d_tab, m_tab, ypT, ytT, gid2)
    return out.reshape(())


# speedup vs baseline: 1.2208x; 1.0035x over previous
import numpy as np
import jax
import jax.numpy as jnp
from jax import lax
from jax.experimental import pallas as pl
from jax.experimental.pallas import tpu as pltpu

_LANES = 128


def _logaddexp(a, b):
    m = jnp.maximum(a, b)
    return m + jnp.log(1.0 + jnp.exp(-jnp.abs(a - b)))


def _kernel_body(d_tab, m_tab, yp_ref, yt_ref, gid_ref, out_ref, accum):
    step = pl.program_id(0)
    num_steps = pl.num_programs(0)
    R = gid_ref.shape[0]

    row_iota = lax.broadcasted_iota(jnp.int32, (R, _LANES), 0)
    lane_iota = lax.broadcasted_iota(jnp.int32, (R, _LANES), 1)
    idx_flat = row_iota * _LANES + lane_iota

    gid = gid_ref[...]
    yt = yt_ref[0]
    s = yp_ref[0]

    bits = lax.bitcast_convert_type(yt, jnp.int32)
    flip = jnp.where(bits < 0, jnp.int32(-1), jnp.int32(-2147483648))
    u = bits ^ flip
    high24 = lax.shift_right_logical(u, 8)
    w1 = lax.shift_left(gid, 24) | high24
    w2 = lax.shift_left(u & jnp.int32(0xFF), 15) | idx_flat

    n_pass = d_tab.shape[0]

    def exchange(i, carry):
        w1c, w2c, sc = carry
        d = d_tab[i]
        m = m_tab[i]
        bit = (idx_flat & d) != 0

        def lane_partners(ops):
            d_f = _LANES - d
            return tuple(
                jnp.where(
                    bit,
                    pltpu.roll(x, d, axis=1),
                    pltpu.roll(x, d_f, axis=1),
                )
                for x in ops
            )

        def sub_partners(ops):
            dr = lax.shift_right_logical(d, 7)
            dr_f = R - dr
            return tuple(
                jnp.where(
                    bit,
                    pltpu.roll(x, dr, axis=0),
                    pltpu.roll(x, dr_f, axis=0),
                )
                for x in ops
            )

        p1, p2, ps = lax.cond(
            d < _LANES, lane_partners, sub_partners, (w1c, w2c, sc)
        )

        self_lt = (w1c < p1) | ((w1c == p1) & (w2c < p2))
        asc = (idx_flat & m) == 0
        want_min = asc != bit
        keep = self_lt == want_min
        return (
            jnp.where(keep, w1c, p1),
            jnp.where(keep, w2c, p2),
            jnp.where(keep, sc, ps),
        )

    w1, w2, s = lax.fori_loop(0, n_pass, exchange, (w1, w2, s), unroll=12)

    sgid = lax.shift_right_logical(w1, 24)
    v = s
    for t in range(15):
        sh = 1 << t
        if sh < _LANES:
            pv = pltpu.roll(v, _LANES - sh, axis=1)
            pv = jnp.where(
                lane_iota < _LANES - sh, pv, pltpu.roll(pv, R - 1, axis=0)
            )
            pg = pltpu.roll(sgid, _LANES - sh, axis=1)
            pg = jnp.where(
                lane_iota < _LANES - sh, pg, pltpu.roll(pg, R - 1, axis=0)
            )
            valid = ~((row_iota == R - 1) & (lane_iota >= _LANES - sh))
        else:
            rs = sh // _LANES
            pv = pltpu.roll(v, R - rs, axis=0)
            pg = pltpu.roll(sgid, R - rs, axis=0)
            valid = row_iota < R - rs
        m_ok = valid & (pg == sgid)
        v = jnp.where(m_ok, _logaddexp(v, pv), v)

    contrib = s - v

    @pl.when(step == 0)
    def _():
        accum[...] = contrib

    @pl.when(step != 0)
    def _():
        accum[...] = accum[...] + contrib

    @pl.when(step == num_steps - 1)
    def _():
        A = accum[...]
        T = A
        C = jnp.full((R, _LANES), 1.0, dtype=jnp.float32)
        for t in range(15):
            sh = 1 << t
            if sh < _LANES:
                pT = pltpu.roll(T, _LANES - sh, axis=1)
                pT = jnp.where(
                    lane_iota < _LANES - sh, pT, pltpu.roll(pT, R - 1, axis=0)
                )
                pC = pltpu.roll(C, _LANES - sh, axis=1)
                pC = jnp.where(
                    lane_iota < _LANES - sh, pC, pltpu.roll(pC, R - 1, axis=0)
                )
                pg = pltpu.roll(gid, _LANES - sh, axis=1)
                pg = jnp.where(
                    lane_iota < _LANES - sh, pg, pltpu.roll(pg, R - 1, axis=0)
                )
                valid = ~((row_iota == R - 1) & (lane_iota >= _LANES - sh))
            else:
                rs = sh // _LANES
                pT = pltpu.roll(T, R - rs, axis=0)
                pC = pltpu.roll(C, R - rs, axis=0)
                pg = pltpu.roll(gid, R - rs, axis=0)
                valid = row_iota < R - rs
            m_ok = valid & (pg == gid)
            T = jnp.where(m_ok, T + pT, T)
            C = jnp.where(m_ok, C + pC, C)

        prev = pltpu.roll(gid, 1, axis=1)
        prev = jnp.where(lane_iota > 0, prev, pltpu.roll(prev, 1, axis=0))
        is_start = (idx_flat == 0) | (gid != prev)
        denom_cols = jnp.float32(num_steps)
        group_mean = jnp.where(is_start, T / (C * denom_cols), 0.0)
        n_groups = jnp.sum(jnp.where(is_start, 1.0, 0.0))
        out_ref[0, 0] = -jnp.sum(group_mean) / n_groups


def kernel(y_pred, y_true, group_ids):
    N, D = y_pred.shape
    R = N // _LANES

    ds, ms = [], []
    m = 2
    while m <= N:
        d = m // 2
        while d >= 1:
            ds.append(d)
            ms.append(m)
            d //= 2
        m *= 2
    d_tab = jnp.asarray(np.array(ds, dtype=np.int32))
    m_tab = jnp.asarray(np.array(ms, dtype=np.int32))

    ypT = (-y_pred).T.reshape(D, R, _LANES)
    ytT = y_true.T.reshape(D, R, _LANES)
    gid2 = group_ids.astype(jnp.int32).reshape(R, _LANES)

    out = pl.pallas_call(
        _kernel_body,
        grid=(D,),
        in_specs=[
            pl.BlockSpec(memory_space=pltpu.SMEM),
            pl.BlockSpec(memory_space=pltpu.SMEM),
            pl.BlockSpec((1, R, _LANES), lambda d: (d, 0, 0)),
            pl.BlockSpec((1, R, _LANES), lambda d: (d, 0, 0)),
            pl.BlockSpec((R, _LANES), lambda d: (0, 0)),
        ],
        out_specs=pl.BlockSpec(memory_space=pltpu.SMEM),
        out_shape=jax.ShapeDtypeStruct((1, 1), jnp.float32),
        scratch_shapes=[pltpu.VMEM((R, _LANES), jnp.float32)],
        compiler_params=pltpu.CompilerParams(
            dimension_semantics=("arbitrary",)
        ),
    )(d_tab, m_tab, ypT, ytT, gid2)
    return out.reshape(())
